# Initial kernel scaffold; baseline (speedup 1.0000x reference)
import functools

import jax
import jax.numpy as jnp
from jax.experimental import pallas as pl
from jax.experimental.pallas import tpu as pltpu

NTA = 65536
NC = 512
NO = 128
HID = 166
NSPE = 4
BLK = 2048


def _silu_grad(x, s):
    return s * (1.0 + x * (1.0 - s))


def _fused_body(sym_ref, pos_ref, cid_ref, emb_ref, Wd_ref, bd_ref,
                W1_ref, b1_ref, W1T_ref, W2r_ref, b2_ref,
                e_ref, f_ref, en_ref):
    pos = pos_ref[:]
    z = (pos[:, 0:1] * Wd_ref[0][None, :]
         + pos[:, 1:2] * Wd_ref[1][None, :]
         + pos[:, 2:3] * Wd_ref[2][None, :]
         + bd_ref[:])
    sig_z = jax.nn.sigmoid(z)
    a = z * sig_z

    sym = sym_ref[:]
    embg = jnp.zeros_like(z)
    for s in range(NSPE):
        m = (sym == s).astype(jnp.float32)
        embg = embg + m * emb_ref[s][None, :]
    d = a * embg

    e_acc = jnp.zeros((d.shape[0], 1), jnp.float32)
    g_d = jnp.zeros_like(d)
    for s in range(NSPE):
        u = jnp.dot(d, W1_ref[s], preferred_element_type=jnp.float32) \
            + b1_ref[s][None, :]
        sig_u = jax.nn.sigmoid(u)
        h = u * sig_u
        w2row = W2r_ref[s]
        es = jnp.sum(h * w2row, axis=1, keepdims=True) + b2_ref[s][None, :]
        m = sym == s
        e_acc = e_acc + jnp.where(m, es, 0.0)
        gu = _silu_grad(u, sig_u) * w2row
        gds = jnp.dot(gu, W1T_ref[s], preferred_element_type=jnp.float32)
        g_d = g_d + jnp.where(m, gds, 0.0)

    g_z = g_d * embg * _silu_grad(z, sig_z)
    f0 = jnp.sum(g_z * Wd_ref[0][None, :], axis=1, keepdims=True)
    f1 = jnp.sum(g_z * Wd_ref[1][None, :], axis=1, keepdims=True)
    f2 = jnp.sum(g_z * Wd_ref[2][None, :], axis=1, keepdims=True)
    f_ref[:] = jnp.concatenate([f0, f1, f2], axis=1)
    e_ref[:] = e_acc

    cid = cid_ref[:]
    onehot = (cid == jax.lax.broadcasted_iota(jnp.int32, (1, NC), 1)
              ).astype(jnp.float32)
    part = jax.lax.dot_general(e_acc, onehot,
                               (((0,), (0,)), ((), ())),
                               preferred_element_type=jnp.float32)

    @pl.when(pl.program_id(0) == 0)
    def _init():
        en_ref[:] = jnp.zeros_like(en_ref)

    en_ref[:] += part


@functools.partial(jax.jit, static_argnames=("interpret",))
def _run(symbols, positions, crystalidx, emb, Wd, bd, W1, b1, W2, b2,
         interpret=False):
    sym2d = symbols.astype(jnp.int32).reshape(NTA, 1)
    cid2d = crystalidx.astype(jnp.int32).reshape(NTA, 1)
    bd2d = bd.reshape(1, NO)
    W1T = jnp.transpose(W1, (0, 2, 1))
    W2r = jnp.transpose(W2, (0, 2, 1))

    grid = (NTA // BLK,)
    full = lambda *shape: pl.BlockSpec(shape, lambda i: (0,) * len(shape))
    e, f, en = pl.pallas_call(
        _fused_body,
        grid=grid,
        in_specs=[
            pl.BlockSpec((BLK, 1), lambda i: (i, 0)),
            pl.BlockSpec((BLK, 3), lambda i: (i, 0)),
            pl.BlockSpec((BLK, 1), lambda i: (i, 0)),
            full(NSPE, NO),
            full(3, NO),
            full(1, NO),
            full(NSPE, NO, HID),
            full(NSPE, HID),
            full(NSPE, HID, NO),
            full(NSPE, 1, HID),
            full(NSPE, 1),
        ],
        out_specs=[
            pl.BlockSpec((BLK, 1), lambda i: (i, 0)),
            pl.BlockSpec((BLK, 3), lambda i: (i, 0)),
            pl.BlockSpec((1, NC), lambda i: (0, 0)),
        ],
        out_shape=[
            jax.ShapeDtypeStruct((NTA, 1), jnp.float32),
            jax.ShapeDtypeStruct((NTA, 3), jnp.float32),
            jax.ShapeDtypeStruct((1, NC), jnp.float32),
        ],
        interpret=interpret,
    )(sym2d, positions, cid2d, emb, Wd, bd2d, W1, b1, W1T, W2r, b2)
    return e[:, 0], en[0], f


def kernel(symbols, positions, cells, pbcs, energyidx, crystalidx,
           emb, Wd, bd, W1, b1, W2, b2):
    return _run(symbols, positions, crystalidx, emb, Wd, bd, W1, b1, W2, b2)


# speedup vs baseline: 3.6731x; 3.6731x over previous
import functools

import jax
import jax.numpy as jnp
from jax.experimental import pallas as pl
from jax.experimental.pallas import tpu as pltpu

NTA = 65536
NC = 512
NO = 128
HID = 166
NSPE = 4
BLK = 2048


def _silu_grad(x, s):
    return s * (1.0 + x * (1.0 - s))


def _b16(x):
    return x.astype(jnp.bfloat16).astype(jnp.float32)


def _fused_body(sym_ref, pos_ref, cid_ref, emb_ref, Wd_ref, bd_ref,
                W1_ref, b1_ref, W1T_ref, W2r_ref, b2_ref,
                e_ref, f_ref, en_ref):
    pos = _b16(pos_ref[:])
    Wdb = _b16(Wd_ref[:])
    z = (pos[:, 0:1] * Wdb[0][None, :]
         + pos[:, 1:2] * Wdb[1][None, :]
         + pos[:, 2:3] * Wdb[2][None, :]
         + bd_ref[:])
    sig_z = jax.nn.sigmoid(z)
    a = z * sig_z

    sym = sym_ref[:]
    embg = jnp.zeros_like(z)
    for s in range(NSPE):
        m = (sym == s).astype(jnp.float32)
        embg = embg + m * emb_ref[s][None, :]
    d = a * embg

    db = d.astype(jnp.bfloat16)
    e_acc = jnp.zeros((d.shape[0], 1), jnp.float32)
    g_d = jnp.zeros_like(d)
    for s in range(NSPE):
        u = jnp.dot(db, W1_ref[s].astype(jnp.bfloat16),
                    preferred_element_type=jnp.float32) \
            + b1_ref[s][None, :]
        sig_u = jax.nn.sigmoid(u)
        h = u * sig_u
        w2row = _b16(W2r_ref[s])
        es = jnp.sum(_b16(h) * w2row, axis=1, keepdims=True) \
            + b2_ref[s][None, :]
        m = sym == s
        e_acc = e_acc + jnp.where(m, es, 0.0)
        gu = _silu_grad(u, sig_u) * w2row
        gds = jnp.dot(gu.astype(jnp.bfloat16),
                      W1T_ref[s].astype(jnp.bfloat16),
                      preferred_element_type=jnp.float32)
        g_d = g_d + jnp.where(m, gds, 0.0)

    g_z = _b16(g_d * embg * _silu_grad(z, sig_z))
    f0 = jnp.sum(g_z * Wdb[0][None, :], axis=1, keepdims=True)
    f1 = jnp.sum(g_z * Wdb[1][None, :], axis=1, keepdims=True)
    f2 = jnp.sum(g_z * Wdb[2][None, :], axis=1, keepdims=True)
    f_ref[:] = jnp.concatenate([f0, f1, f2], axis=1)
    e_ref[:] = e_acc

    cid = cid_ref[:]
    onehot = (cid == jax.lax.broadcasted_iota(jnp.int32, (1, NC), 1)
              ).astype(jnp.float32)
    part = jax.lax.dot_general(e_acc, onehot,
                               (((0,), (0,)), ((), ())),
                               preferred_element_type=jnp.float32)

    @pl.when(pl.program_id(0) == 0)
    def _init():
        en_ref[:] = jnp.zeros_like(en_ref)

    en_ref[:] += part


@functools.partial(jax.jit, static_argnames=("interpret",))
def _run(symbols, positions, crystalidx, emb, Wd, bd, W1, b1, W2, b2,
         interpret=False):
    sym2d = symbols.astype(jnp.int32).reshape(NTA, 1)
    cid2d = crystalidx.astype(jnp.int32).reshape(NTA, 1)
    bd2d = bd.reshape(1, NO)
    W1T = jnp.transpose(W1, (0, 2, 1))
    W2r = jnp.transpose(W2, (0, 2, 1))

    grid = (NTA // BLK,)
    full = lambda *shape: pl.BlockSpec(shape, lambda i: (0,) * len(shape))
    e, f, en = pl.pallas_call(
        _fused_body,
        grid=grid,
        in_specs=[
            pl.BlockSpec((BLK, 1), lambda i: (i, 0)),
            pl.BlockSpec((BLK, 3), lambda i: (i, 0)),
            pl.BlockSpec((BLK, 1), lambda i: (i, 0)),
            full(NSPE, NO),
            full(3, NO),
            full(1, NO),
            full(NSPE, NO, HID),
            full(NSPE, HID),
            full(NSPE, HID, NO),
            full(NSPE, 1, HID),
            full(NSPE, 1),
        ],
        out_specs=[
            pl.BlockSpec((BLK, 1), lambda i: (i, 0)),
            pl.BlockSpec((BLK, 3), lambda i: (i, 0)),
            pl.BlockSpec((1, NC), lambda i: (0, 0)),
        ],
        out_shape=[
            jax.ShapeDtypeStruct((NTA, 1), jnp.float32),
            jax.ShapeDtypeStruct((NTA, 3), jnp.float32),
            jax.ShapeDtypeStruct((1, NC), jnp.float32),
        ],
        interpret=interpret,
    )(sym2d, positions, cid2d, emb, Wd, bd2d, W1, b1, W1T, W2r, b2)
    return e[:, 0], en[0], f


def kernel(symbols, positions, cells, pbcs, energyidx, crystalidx,
           emb, Wd, bd, W1, b1, W2, b2):
    return _run(symbols, positions, crystalidx, emb, Wd, bd, W1, b1, W2, b2)


# speedup vs baseline: 4.0088x; 1.0914x over previous
import functools

import jax
import jax.numpy as jnp
from jax.experimental import pallas as pl
from jax.experimental.pallas import tpu as pltpu

NTA = 65536
NC = 512
NO = 128
HID = 166
NSPE = 4
BLK = 2048


def _silu_grad(x, s):
    return s * (1.0 + x * (1.0 - s))


def _b16(x):
    return x.astype(jnp.bfloat16).astype(jnp.float32)


def _fused_body(sym_ref, pos_ref, cid_ref, emb_ref, Wd_ref, bd_ref,
                W1_ref, b1_ref, W1T_ref, W2r_ref, b2_ref,
                e_ref, f_ref, en_ref):
    pos = _b16(pos_ref[:])
    Wdb = _b16(Wd_ref[:])
    z = (pos[:, 0:1] * Wdb[0][None, :]
         + pos[:, 1:2] * Wdb[1][None, :]
         + pos[:, 2:3] * Wdb[2][None, :]
         + bd_ref[:])
    sig_z = jax.nn.sigmoid(z)
    a = z * sig_z

    sym = sym_ref[:]
    embg = jnp.zeros_like(z)
    for s in range(NSPE):
        m = (sym == s).astype(jnp.float32)
        embg = embg + m * emb_ref[s][None, :]
    d = a * embg

    oh4 = (sym == jax.lax.broadcasted_iota(jnp.int32, (1, NSPE), 1)
           ).astype(jnp.float32)
    b1g = jnp.dot(oh4, b1_ref[:], preferred_element_type=jnp.float32)
    w2g = _b16(jnp.dot(oh4, W2r_ref[:, 0, :],
                       preferred_element_type=jnp.float32))
    b2g = jnp.dot(oh4, b2_ref[:], preferred_element_type=jnp.float32)

    db = d.astype(jnp.bfloat16)
    zb = jnp.zeros_like(db)
    u = b1g
    for s in range(NSPE):
        dm = jnp.where(sym == s, db, zb)
        u = u + jnp.dot(dm, W1_ref[s].astype(jnp.bfloat16),
                        preferred_element_type=jnp.float32)
    sig_u = jax.nn.sigmoid(u)
    h = u * sig_u
    e_acc = jnp.sum(_b16(h) * w2g, axis=1, keepdims=True) + b2g

    gu = (_silu_grad(u, sig_u) * w2g).astype(jnp.bfloat16)
    zh = jnp.zeros_like(gu)
    g_d = jnp.zeros_like(d)
    for s in range(NSPE):
        gm = jnp.where(sym == s, gu, zh)
        g_d = g_d + jnp.dot(gm, W1T_ref[s].astype(jnp.bfloat16),
                            preferred_element_type=jnp.float32)

    g_z = _b16(g_d * embg * _silu_grad(z, sig_z))
    f0 = jnp.sum(g_z * Wdb[0][None, :], axis=1, keepdims=True)
    f1 = jnp.sum(g_z * Wdb[1][None, :], axis=1, keepdims=True)
    f2 = jnp.sum(g_z * Wdb[2][None, :], axis=1, keepdims=True)
    f_ref[:] = jnp.concatenate([f0, f1, f2], axis=1)
    e_ref[:] = e_acc

    cid = cid_ref[:]
    onehot = (cid == jax.lax.broadcasted_iota(jnp.int32, (1, NC), 1)
              ).astype(jnp.float32)
    part = jax.lax.dot_general(e_acc, onehot,
                               (((0,), (0,)), ((), ())),
                               preferred_element_type=jnp.float32)

    @pl.when(pl.program_id(0) == 0)
    def _init():
        en_ref[:] = jnp.zeros_like(en_ref)

    en_ref[:] += part


@functools.partial(jax.jit, static_argnames=("interpret",))
def _run(symbols, positions, crystalidx, emb, Wd, bd, W1, b1, W2, b2,
         interpret=False):
    sym2d = symbols.astype(jnp.int32).reshape(NTA, 1)
    cid2d = crystalidx.astype(jnp.int32).reshape(NTA, 1)
    bd2d = bd.reshape(1, NO)
    W1T = jnp.transpose(W1, (0, 2, 1))
    W2r = jnp.transpose(W2, (0, 2, 1))

    grid = (NTA // BLK,)
    full = lambda *shape: pl.BlockSpec(shape, lambda i: (0,) * len(shape))
    e, f, en = pl.pallas_call(
        _fused_body,
        grid=grid,
        in_specs=[
            pl.BlockSpec((BLK, 1), lambda i: (i, 0)),
            pl.BlockSpec((BLK, 3), lambda i: (i, 0)),
            pl.BlockSpec((BLK, 1), lambda i: (i, 0)),
            full(NSPE, NO),
            full(3, NO),
            full(1, NO),
            full(NSPE, NO, HID),
            full(NSPE, HID),
            full(NSPE, HID, NO),
            full(NSPE, 1, HID),
            full(NSPE, 1),
        ],
        out_specs=[
            pl.BlockSpec((BLK, 1), lambda i: (i, 0)),
            pl.BlockSpec((BLK, 3), lambda i: (i, 0)),
            pl.BlockSpec((1, NC), lambda i: (0, 0)),
        ],
        out_shape=[
            jax.ShapeDtypeStruct((NTA, 1), jnp.float32),
            jax.ShapeDtypeStruct((NTA, 3), jnp.float32),
            jax.ShapeDtypeStruct((1, NC), jnp.float32),
        ],
        interpret=interpret,
    )(sym2d, positions, cid2d, emb, Wd, bd2d, W1, b1, W1T, W2r, b2)
    return e[:, 0], en[0], f


def kernel(symbols, positions, cells, pbcs, energyidx, crystalidx,
           emb, Wd, bd, W1, b1, W2, b2):
    return _run(symbols, positions, crystalidx, emb, Wd, bd, W1, b1, W2, b2)


# speedup vs baseline: 4.5873x; 1.1443x over previous
import functools

import jax
import jax.numpy as jnp
from jax.experimental import pallas as pl
from jax.experimental.pallas import tpu as pltpu

NTA = 65536
NC = 512
NO = 128
HID = 166
NSPE = 4
BLK = 2048


def _silu_grad(x, s):
    return s * (1.0 + x * (1.0 - s))


def _b16(x):
    return x.astype(jnp.bfloat16).astype(jnp.float32)


def _fused_body(sym_ref, pos_ref, cid_ref, emb_ref, Wd_ref, WdT_ref, bd_ref,
                W1_ref, b1_ref, W1T_ref, W2r_ref, b2_ref,
                e_ref, f_ref, en_ref):
    z = jnp.dot(pos_ref[:].astype(jnp.bfloat16),
                Wd_ref[:].astype(jnp.bfloat16),
                preferred_element_type=jnp.float32) + bd_ref[:]
    sig_z = jax.nn.sigmoid(z)
    a = z * sig_z

    sym = sym_ref[:]
    oh4 = (sym == jax.lax.broadcasted_iota(jnp.int32, (1, NSPE), 1)
           ).astype(jnp.float32)
    embg = jnp.dot(oh4, emb_ref[:], preferred_element_type=jnp.float32)
    b1g = jnp.dot(oh4, b1_ref[:], preferred_element_type=jnp.float32)
    w2g = _b16(jnp.dot(oh4, W2r_ref[:, 0, :],
                       preferred_element_type=jnp.float32))
    b2g = jnp.dot(oh4, b2_ref[:], preferred_element_type=jnp.float32)
    d = a * embg

    db = d.astype(jnp.bfloat16)
    zb = jnp.zeros_like(db)
    u = b1g
    for s in range(NSPE):
        dm = jnp.where(sym == s, db, zb)
        u = u + jnp.dot(dm, W1_ref[s].astype(jnp.bfloat16),
                        preferred_element_type=jnp.float32)
    sig_u = jax.nn.sigmoid(u)
    h = u * sig_u
    e_acc = jnp.sum(_b16(h) * w2g, axis=1, keepdims=True) + b2g

    gu = (_silu_grad(u, sig_u) * w2g).astype(jnp.bfloat16)
    zh = jnp.zeros_like(gu)
    g_d = jnp.zeros_like(d)
    for s in range(NSPE):
        gm = jnp.where(sym == s, gu, zh)
        g_d = g_d + jnp.dot(gm, W1T_ref[s].astype(jnp.bfloat16),
                            preferred_element_type=jnp.float32)

    g_z = (g_d * embg * _silu_grad(z, sig_z)).astype(jnp.bfloat16)
    f_ref[:] = jnp.dot(g_z, WdT_ref[:].astype(jnp.bfloat16),
                       preferred_element_type=jnp.float32)
    e_ref[:] = e_acc

    cid = cid_ref[:]
    onehot = (cid == jax.lax.broadcasted_iota(jnp.int32, (1, NC), 1)
              ).astype(jnp.float32)
    part = jax.lax.dot_general(e_acc, onehot,
                               (((0,), (0,)), ((), ())),
                               preferred_element_type=jnp.float32)

    @pl.when(pl.program_id(0) == 0)
    def _init():
        en_ref[:] = jnp.zeros_like(en_ref)

    en_ref[:] += part


@functools.partial(jax.jit, static_argnames=("interpret",))
def _run(symbols, positions, crystalidx, emb, Wd, bd, W1, b1, W2, b2,
         interpret=False):
    sym2d = symbols.astype(jnp.int32).reshape(NTA, 1)
    cid2d = crystalidx.astype(jnp.int32).reshape(NTA, 1)
    bd2d = bd.reshape(1, NO)
    WdT = jnp.transpose(Wd)
    W1T = jnp.transpose(W1, (0, 2, 1))
    W2r = jnp.transpose(W2, (0, 2, 1))

    grid = (NTA // BLK,)
    full = lambda *shape: pl.BlockSpec(shape, lambda i: (0,) * len(shape))
    e, f, en = pl.pallas_call(
        _fused_body,
        grid=grid,
        in_specs=[
            pl.BlockSpec((BLK, 1), lambda i: (i, 0)),
            pl.BlockSpec((BLK, 3), lambda i: (i, 0)),
            pl.BlockSpec((BLK, 1), lambda i: (i, 0)),
            full(NSPE, NO),
            full(3, NO),
            full(NO, 3),
            full(1, NO),
            full(NSPE, NO, HID),
            full(NSPE, HID),
            full(NSPE, HID, NO),
            full(NSPE, 1, HID),
            full(NSPE, 1),
        ],
        out_specs=[
            pl.BlockSpec((BLK, 1), lambda i: (i, 0)),
            pl.BlockSpec((BLK, 3), lambda i: (i, 0)),
            pl.BlockSpec((1, NC), lambda i: (0, 0)),
        ],
        out_shape=[
            jax.ShapeDtypeStruct((NTA, 1), jnp.float32),
            jax.ShapeDtypeStruct((NTA, 3), jnp.float32),
            jax.ShapeDtypeStruct((1, NC), jnp.float32),
        ],
        interpret=interpret,
    )(sym2d, positions, cid2d, emb, Wd, WdT, bd2d, W1, b1, W1T, W2r, b2)
    return e[:, 0], en[0], f


def kernel(symbols, positions, cells, pbcs, energyidx, crystalidx,
           emb, Wd, bd, W1, b1, W2, b2):
    return _run(symbols, positions, crystalidx, emb, Wd, bd, W1, b1, W2, b2)
